# Initial kernel scaffold; baseline (speedup 1.0000x reference)
import functools

import jax
import jax.numpy as jnp
from jax import lax
from jax.experimental import pallas as pl
from jax.experimental.pallas import tpu as pltpu
from jax.experimental.pallas import tpu_sc as plsc

N_NODES = 10000
N_EDGES = 320000
F = 128
EA = 16

NW = 32
CHUNK = 128
CPT = 79
EDGES_PAD = NW * CPT * CHUNK
TILE_EDGES = CPT * CHUNK
NODES_PAD = 10240
ACC_ROWS = 104
ACC_W = 144


def _t1_body(x_ref, wa_ref, wb_ref, a_ref, b_ref):
    xv = x_ref[...]
    dn = (((1,), (1,)), ((), ()))
    a_ref[...] = lax.dot_general(xv, wa_ref[...], dn,
                                 preferred_element_type=jnp.float32)
    b_ref[...] = lax.dot_general(xv, wb_ref[...], dn,
                                 preferred_element_type=jnp.float32)


def _t2_body(ea_ref, wc_ref, b1_ref, gi_ref, c_ref, g_ref):
    dn = (((1,), (1,)), ((), ()))
    c_ref[...] = lax.dot_general(ea_ref[...], wc_ref[...], dn,
                                 preferred_element_type=jnp.float32) + b1_ref[...]
    gi = gi_ref[...].astype(jnp.float32)
    g_ref[...] = jnp.floor((gi + 0.5) * 0.01).astype(jnp.int32)


def _t4_body(acc_ref, x3_ref, w2_ref, b2_ref, l1a_ref, l1b_ref, l1bias_ref,
             gvw_ref, gvb_ref, out_ref):
    acc = acc_ref[...]
    rsum = acc[0, :100, :128] + acc[1, :100, :128]
    deg = acc[0, :100, 128:129] + acc[1, :100, 128:129]
    dn = (((1,), (1,)), ((), ()))
    g = lax.dot_general(rsum, w2_ref[...], dn,
                        preferred_element_type=jnp.float32) + deg * b2_ref[...]
    xs = jnp.sum(x3_ref[...], axis=1)
    v = lax.dot_general(xs, l1a_ref[...], dn, preferred_element_type=jnp.float32)
    v = v + lax.dot_general(g, l1b_ref[...], dn, preferred_element_type=jnp.float32)
    v = jnp.maximum(v + l1bias_ref[...], 0.0)
    out_ref[...] = lax.dot_general(v, gvw_ref[...], dn,
                                   preferred_element_type=jnp.float32) + gvb_ref[...]


def _sc_edge_body(a_hbm, b_hbm, c_hbm, ii_hbm, jj_hbm, gg_hbm, out_hbm,
                  gi_v, gj_v, gg_v, buf_a, buf_b, buf_c, buf_r, acc,
                  sem0, sem1):
    cid = lax.axis_index("c")
    tid = lax.axis_index("s")
    wid = cid * 16 + tid
    base_w = wid * TILE_EDGES

    zero16 = jnp.zeros((16,), jnp.float32)
    head = (lax.iota(jnp.int32, 16) == 0).astype(jnp.float32)

    @pl.when(tid == 0)
    def _():
        def zrow(r, carry):
            for k in range(ACC_W // 16):
                buf_r[r, pl.ds(k * 16, 16)] = zero16
            return carry
        lax.fori_loop(0, ACC_ROWS, zrow, 0)
        pltpu.sync_copy(buf_r.at[pl.ds(0, ACC_ROWS)], acc)

    plsc.subcore_barrier()

    def chunk_body(c, carry):
        base = base_w + c * CHUNK
        pltpu.sync_copy(ii_hbm.at[pl.ds(base, CHUNK)], gi_v)
        pltpu.sync_copy(jj_hbm.at[pl.ds(base, CHUNK)], gj_v)
        pltpu.sync_copy(gg_hbm.at[pl.ds(base, CHUNK)], gg_v)
        da = pltpu.async_copy(a_hbm.at[gi_v], buf_a, sem0)
        db = pltpu.async_copy(b_hbm.at[gj_v], buf_b, sem1)
        pltpu.sync_copy(c_hbm.at[pl.ds(base, CHUNK)], buf_c)
        da.wait()
        db.wait()

        def edge_body(e, ecarry):
            for k in range(8):
                s = pl.ds(k * 16, 16)
                v = buf_a[e, s] + buf_b[e, s] + buf_c[e, s]
                buf_r[e, s] = jnp.maximum(v, 0.0)
            buf_r[e, pl.ds(128, 16)] = head
            return ecarry
        lax.fori_loop(0, CHUNK, edge_body, 0)

        pltpu.sync_copy(buf_r, acc.at[gg_v], add=True)
        return carry

    lax.fori_loop(0, CPT, chunk_body, 0)

    plsc.subcore_barrier()

    @pl.when(tid == 0)
    def _():
        pltpu.sync_copy(acc, out_hbm.at[cid])


_sc_edge_kernel = functools.partial(
    pl.kernel,
    out_type=jax.ShapeDtypeStruct((2, ACC_ROWS, ACC_W), jnp.float32),
    mesh=plsc.VectorSubcoreMesh(core_axis_name="c", subcore_axis_name="s"),
    scratch_types=[
        pltpu.VMEM((CHUNK,), jnp.int32),
        pltpu.VMEM((CHUNK,), jnp.int32),
        pltpu.VMEM((CHUNK,), jnp.int32),
        pltpu.VMEM((CHUNK, F), jnp.float32),
        pltpu.VMEM((CHUNK, F), jnp.float32),
        pltpu.VMEM((CHUNK, F), jnp.float32),
        pltpu.VMEM((CHUNK, ACC_W), jnp.float32),
        pltpu.VMEM_SHARED((ACC_ROWS, ACC_W), jnp.float32),
        pltpu.SemaphoreType.DMA,
        pltpu.SemaphoreType.DMA,
    ],
)(_sc_edge_body)


@jax.jit
def kernel(x, edge_index, edge_attr, W1, b1, W2, b2, lin1_W, lin1_b, gv_W, gv_b):
    f32 = jnp.float32
    idx_i = edge_index[0].astype(jnp.int32)
    idx_j = edge_index[1].astype(jnp.int32)
    npad = EDGES_PAD - N_EDGES
    ii = jnp.concatenate([idx_i, jnp.full((npad,), N_NODES, jnp.int32)])
    jj = jnp.concatenate([idx_j, jnp.zeros((npad,), jnp.int32)])
    gi2d = ii.reshape(EDGES_PAD // 128, 128)
    ea_pad = jnp.concatenate([edge_attr, jnp.zeros((npad, EA), f32)])
    x_pad = jnp.concatenate([x, jnp.zeros((NODES_PAD - N_NODES, F), f32)])

    W1a = W1[:, :F]
    W1b = W1[:, F:2 * F]
    W1c = W1[:, 2 * F:]
    b1r = b1.reshape(1, F)

    a_tab, b_tab = pl.pallas_call(
        _t1_body,
        grid=(10,),
        in_specs=[
            pl.BlockSpec((1024, F), lambda i: (i, 0)),
            pl.BlockSpec((F, F), lambda i: (0, 0)),
            pl.BlockSpec((F, F), lambda i: (0, 0)),
        ],
        out_specs=[
            pl.BlockSpec((1024, F), lambda i: (i, 0)),
            pl.BlockSpec((1024, F), lambda i: (i, 0)),
        ],
        out_shape=[
            jax.ShapeDtypeStruct((NODES_PAD, F), f32),
            jax.ShapeDtypeStruct((NODES_PAD, F), f32),
        ],
    )(x_pad, W1a, W1b)

    c_tab, g2d = pl.pallas_call(
        _t2_body,
        grid=(EDGES_PAD // 1024,),
        in_specs=[
            pl.BlockSpec((1024, EA), lambda i: (i, 0)),
            pl.BlockSpec((F, EA), lambda i: (0, 0)),
            pl.BlockSpec((1, F), lambda i: (0, 0)),
            pl.BlockSpec((8, 128), lambda i: (i, 0)),
        ],
        out_specs=[
            pl.BlockSpec((1024, F), lambda i: (i, 0)),
            pl.BlockSpec((8, 128), lambda i: (i, 0)),
        ],
        out_shape=[
            jax.ShapeDtypeStruct((EDGES_PAD, F), f32),
            jax.ShapeDtypeStruct((EDGES_PAD // 128, 128), jnp.int32),
        ],
    )(ea_pad, W1c, b1r, gi2d)

    gg = g2d.reshape(EDGES_PAD)

    acc = _sc_edge_kernel(a_tab, b_tab, c_tab, ii, jj, gg)

    x3 = x.reshape(100, 100, F)
    out = pl.pallas_call(
        _t4_body,
        in_specs=[pl.BlockSpec(memory_space=pltpu.MemorySpace.VMEM)] * 9,
        out_specs=pl.BlockSpec(memory_space=pltpu.MemorySpace.VMEM),
        out_shape=jax.ShapeDtypeStruct((100, 1), f32),
    )(acc, x3, W2, b2.reshape(1, F), lin1_W[:, :F], lin1_W[:, F:],
      lin1_b.reshape(1, F), gv_W, gv_b.reshape(1, 1))

    return out[:, 0]


# speedup vs baseline: 2.2117x; 2.2117x over previous
import functools

import jax
import jax.numpy as jnp
from jax import lax
from jax.experimental import pallas as pl
from jax.experimental.pallas import tpu as pltpu
from jax.experimental.pallas import tpu_sc as plsc

N_NODES = 10000
N_EDGES = 320000
F = 128
EA = 16

NW = 32
CHUNK = 128
CPT = 79
EDGES_PAD = NW * CPT * CHUNK
TILE_EDGES = CPT * CHUNK
NODES_PAD = 10240
ACC_ROWS = 104


def _t1_body(x_ref, wa_ref, wb_ref, a_ref, b_ref):
    xv = x_ref[...]
    dn = (((1,), (1,)), ((), ()))
    a_ref[...] = lax.dot_general(xv, wa_ref[...], dn,
                                 preferred_element_type=jnp.float32)
    b_ref[...] = lax.dot_general(xv, wb_ref[...], dn,
                                 preferred_element_type=jnp.float32)


def _t2_body(ea_ref, wc_ref, b1_ref, gi_ref, c_ref, g_ref, deg_ref):
    dn = (((1,), (1,)), ((), ()))
    c_ref[...] = lax.dot_general(ea_ref[...], wc_ref[...], dn,
                                 preferred_element_type=jnp.float32) + b1_ref[...]
    gi = gi_ref[...].astype(jnp.float32)
    g = jnp.floor((gi + 0.5) * 0.01).astype(jnp.int32)
    g_ref[...] = g
    oh = (lax.broadcasted_iota(jnp.int32, (128, 8, 128), 0)
          == g[None, :, :]).astype(jnp.float32)
    part = jnp.sum(oh, axis=(1, 2)).reshape(128, 1)

    @pl.when(pl.program_id(0) == 0)
    def _():
        deg_ref[...] = part

    @pl.when(pl.program_id(0) != 0)
    def _():
        deg_ref[...] = deg_ref[...] + part


def _t4_body(acc_ref, deg_ref, x3_ref, w2_ref, b2_ref, l1a_ref, l1b_ref,
             l1bias_ref, gvw_ref, gvb_ref, out_ref):
    acc = jnp.sum(acc_ref[...], axis=0)
    rsum = acc[:100, :]
    deg = deg_ref[...][:100, :]
    dn = (((1,), (1,)), ((), ()))
    g = lax.dot_general(rsum, w2_ref[...], dn,
                        preferred_element_type=jnp.float32) + deg * b2_ref[...]
    xs = jnp.sum(x3_ref[...], axis=1)
    v = lax.dot_general(xs, l1a_ref[...], dn, preferred_element_type=jnp.float32)
    v = v + lax.dot_general(g, l1b_ref[...], dn, preferred_element_type=jnp.float32)
    v = jnp.maximum(v + l1bias_ref[...], 0.0)
    out_ref[...] = lax.dot_general(v, gvw_ref[...], dn,
                                   preferred_element_type=jnp.float32) + gvb_ref[0, 0]


def _sc_edge_body(a_hbm, b_hbm, c_hbm, ii_hbm, jj_hbm, gg_hbm, out_hbm,
                  gi_v, gj_v, gg_v, buf_a, buf_b, buf_c, acc_t, sem0, sem1):
    cid = lax.axis_index("c")
    tid = lax.axis_index("s")
    wid = cid * 16 + tid
    base_w = wid * TILE_EDGES

    zero16 = jnp.zeros((16,), jnp.float32)

    def zrow(r, carry):
        for k in range(8):
            acc_t[r, pl.ds(k * 16, 16)] = zero16
        return carry
    lax.fori_loop(0, ACC_ROWS, zrow, 0)

    def chunk_body(c, carry):
        base = base_w + c * CHUNK
        pltpu.sync_copy(ii_hbm.at[pl.ds(base, CHUNK)], gi_v)
        pltpu.sync_copy(jj_hbm.at[pl.ds(base, CHUNK)], gj_v)
        pltpu.sync_copy(gg_hbm.at[pl.ds(base, CHUNK)], gg_v)
        da = pltpu.async_copy(a_hbm.at[gi_v], buf_a, sem0)
        db = pltpu.async_copy(b_hbm.at[gj_v], buf_b, sem1)
        pltpu.sync_copy(c_hbm.at[pl.ds(base, CHUNK)], buf_c)
        da.wait()
        db.wait()

        def q_body(q, qcarry):
            gvec = gg_v[pl.ds(q * 16, 16)]
            for l in range(16):
                e = q * 16 + l
                g = gvec[l]
                for k in range(8):
                    s = pl.ds(k * 16, 16)
                    v = buf_a[e, s] + buf_b[e, s] + buf_c[e, s]
                    plsc.addupdate(acc_t.at[g, s], jnp.maximum(v, 0.0))
            return qcarry
        lax.fori_loop(0, CHUNK // 16, q_body, 0)
        return carry

    lax.fori_loop(0, CPT, chunk_body, 0)

    pltpu.sync_copy(acc_t, out_hbm.at[cid, tid])


@functools.lru_cache(maxsize=1)
def _sc_edge_kernel():
    return functools.partial(
        pl.kernel,
        out_type=jax.ShapeDtypeStruct((2, 16, ACC_ROWS, F), jnp.float32),
        mesh=plsc.VectorSubcoreMesh(core_axis_name="c", subcore_axis_name="s"),
        scratch_types=[
            pltpu.VMEM((CHUNK,), jnp.int32),
            pltpu.VMEM((CHUNK,), jnp.int32),
            pltpu.VMEM((CHUNK,), jnp.int32),
            pltpu.VMEM((CHUNK, F), jnp.float32),
            pltpu.VMEM((CHUNK, F), jnp.float32),
            pltpu.VMEM((CHUNK, F), jnp.float32),
            pltpu.VMEM((ACC_ROWS, F), jnp.float32),
            pltpu.SemaphoreType.DMA,
            pltpu.SemaphoreType.DMA,
        ],
    )(_sc_edge_body)


@jax.jit
def kernel(x, edge_index, edge_attr, W1, b1, W2, b2, lin1_W, lin1_b, gv_W, gv_b):
    f32 = jnp.float32
    idx_i = edge_index[0].astype(jnp.int32)
    idx_j = edge_index[1].astype(jnp.int32)
    npad = EDGES_PAD - N_EDGES
    ii = jnp.concatenate([idx_i, jnp.full((npad,), N_NODES, jnp.int32)])
    jj = jnp.concatenate([idx_j, jnp.zeros((npad,), jnp.int32)])
    gi2d = ii.reshape(EDGES_PAD // 128, 128)
    bf16 = jnp.bfloat16
    ea_pad = jnp.concatenate([edge_attr, jnp.zeros((npad, EA), f32)]).astype(bf16)
    x_pad = jnp.concatenate([x, jnp.zeros((NODES_PAD - N_NODES, F), f32)]).astype(bf16)

    W1a = W1[:, :F].astype(bf16)
    W1b = W1[:, F:2 * F].astype(bf16)
    W1c = W1[:, 2 * F:].astype(bf16)
    b1r = b1.reshape(1, F)

    a_tab, b_tab = pl.pallas_call(
        _t1_body,
        grid=(10,),
        in_specs=[
            pl.BlockSpec((1024, F), lambda i: (i, 0)),
            pl.BlockSpec((F, F), lambda i: (0, 0)),
            pl.BlockSpec((F, F), lambda i: (0, 0)),
        ],
        out_specs=[
            pl.BlockSpec((1024, F), lambda i: (i, 0)),
            pl.BlockSpec((1024, F), lambda i: (i, 0)),
        ],
        out_shape=[
            jax.ShapeDtypeStruct((NODES_PAD, F), f32),
            jax.ShapeDtypeStruct((NODES_PAD, F), f32),
        ],
    )(x_pad, W1a, W1b)

    c_tab, g2d, deg = pl.pallas_call(
        _t2_body,
        grid=(EDGES_PAD // 1024,),
        in_specs=[
            pl.BlockSpec((1024, EA), lambda i: (i, 0)),
            pl.BlockSpec((F, EA), lambda i: (0, 0)),
            pl.BlockSpec((1, F), lambda i: (0, 0)),
            pl.BlockSpec((8, 128), lambda i: (i, 0)),
        ],
        out_specs=[
            pl.BlockSpec((1024, F), lambda i: (i, 0)),
            pl.BlockSpec((8, 128), lambda i: (i, 0)),
            pl.BlockSpec((128, 1), lambda i: (0, 0)),
        ],
        out_shape=[
            jax.ShapeDtypeStruct((EDGES_PAD, F), f32),
            jax.ShapeDtypeStruct((EDGES_PAD // 128, 128), jnp.int32),
            jax.ShapeDtypeStruct((128, 1), f32),
        ],
    )(ea_pad, W1c, b1r, gi2d)

    gg = g2d.reshape(EDGES_PAD)

    acc = _sc_edge_kernel()(a_tab, b_tab, c_tab, ii, jj, gg)
    acc32 = acc.reshape(NW, ACC_ROWS, F)

    x3 = x.reshape(100, 100, F)
    out = pl.pallas_call(
        _t4_body,
        in_specs=[pl.BlockSpec(memory_space=pltpu.MemorySpace.VMEM)] * 9
        + [pl.BlockSpec(memory_space=pltpu.MemorySpace.SMEM)],
        out_specs=pl.BlockSpec(memory_space=pltpu.MemorySpace.VMEM),
        out_shape=jax.ShapeDtypeStruct((100, 8), f32),
    )(acc32, deg, x3, W2.astype(bf16).astype(f32), b2.reshape(1, F),
      lin1_W[:, :F], lin1_W[:, F:],
      lin1_b.reshape(1, F), jnp.concatenate([gv_W, jnp.zeros((7, F), f32)]),
      gv_b.reshape(1, 1))

    return out[:, 0]


# speedup vs baseline: 3.1021x; 1.4026x over previous
import functools

import jax
import jax.numpy as jnp
from jax import lax
from jax.experimental import pallas as pl
from jax.experimental.pallas import tpu as pltpu
from jax.experimental.pallas import tpu_sc as plsc

N_NODES = 10000
N_EDGES = 320000
F = 128
EA = 16

NW = 32
CHUNK = 128
CPT = 80
EDGES_PAD = NW * CPT * CHUNK
TILE_EDGES = CPT * CHUNK
NODES_PAD = 10240
ACC_ROWS = 104


def _t1_body(x_ref, wa_ref, wb_ref, a_ref, b_ref):
    xv = x_ref[...]
    dn = (((1,), (1,)), ((), ()))
    a_ref[...] = lax.dot_general(xv, wa_ref[...], dn,
                                 preferred_element_type=jnp.float32)
    b_ref[...] = lax.dot_general(xv, wb_ref[...], dn,
                                 preferred_element_type=jnp.float32)


def _t2_body(ea_ref, wc_ref, b1_ref, gi_ref, c_ref, deg_ref):
    dn = (((1,), (1,)), ((), ()))
    c_ref[...] = lax.dot_general(ea_ref[...], wc_ref[...], dn,
                                 preferred_element_type=jnp.float32) + b1_ref[...]
    gi = gi_ref[...].astype(jnp.float32)
    g = jnp.floor((gi + 0.5) * 0.01).astype(jnp.int32)
    oh = (lax.broadcasted_iota(jnp.int32, (128, 8, 128), 0)
          == g[None, :, :]).astype(jnp.float32)
    part = jnp.sum(oh, axis=(1, 2)).reshape(128, 1)

    @pl.when(pl.program_id(0) == 0)
    def _():
        deg_ref[...] = part

    @pl.when(pl.program_id(0) != 0)
    def _():
        deg_ref[...] = deg_ref[...] + part


def _t4_body(acc_ref, deg_ref, x3_ref, w2_ref, b2_ref, l1a_ref, l1b_ref,
             l1bias_ref, gvw_ref, gvb_ref, out_ref):
    acc = jnp.sum(acc_ref[...], axis=0)
    rsum = acc[:100, :]
    deg = deg_ref[...][:100, :]
    dn = (((1,), (1,)), ((), ()))
    g = lax.dot_general(rsum, w2_ref[...], dn,
                        preferred_element_type=jnp.float32) + deg * b2_ref[...]
    xs = jnp.sum(x3_ref[...], axis=1)
    v = lax.dot_general(xs, l1a_ref[...], dn, preferred_element_type=jnp.float32)
    v = v + lax.dot_general(g, l1b_ref[...], dn, preferred_element_type=jnp.float32)
    v = jnp.maximum(v + l1bias_ref[...], 0.0)
    out_ref[...] = lax.dot_general(v, gvw_ref[...], dn,
                                   preferred_element_type=jnp.float32) + gvb_ref[0, 0]


def _sc_edge_body(a_hbm, b_hbm, c_hbm, ii_hbm, jj_hbm, out_hbm,
                  i0, j0, i1, j1, g0, g1, a0, b0, c0, a1, b1, c1, acc_t,
                  sa0, sb0, sc0, sa1, sb1, sc1, si0, sj0, si1, sj1):
    cid = lax.axis_index("c")
    tid = lax.axis_index("s")
    wid = cid * 16 + tid
    base_w = wid * TILE_EDGES

    zero16 = jnp.zeros((16,), jnp.float32)

    def zrow(r, carry):
        for k in range(8):
            acc_t[r, pl.ds(k * 16, 16)] = zero16
        return carry
    lax.fori_loop(0, ACC_ROWS, zrow, 0)

    def launch_idx(c, ib, jb, si, sj):
        base = base_w + c * CHUNK
        pltpu.async_copy(ii_hbm.at[pl.ds(base, CHUNK)], ib, si)
        pltpu.async_copy(jj_hbm.at[pl.ds(base, CHUNK)], jb, sj)

    def wait_idx(c, ib, jb, si, sj):
        base = base_w + c * CHUNK
        pltpu.make_async_copy(ii_hbm.at[pl.ds(base, CHUNK)], ib, si).wait()
        pltpu.make_async_copy(jj_hbm.at[pl.ds(base, CHUNK)], jb, sj).wait()

    def launch_gather(c, ib, jb, ba, bb, bc, sa, sb, sc):
        base = base_w + c * CHUNK
        pltpu.async_copy(a_hbm.at[ib], ba, sa)
        pltpu.async_copy(b_hbm.at[jb], bb, sb)
        pltpu.async_copy(c_hbm.at[pl.ds(base, CHUNK)], bc, sc)

    def wait_gather(c, ib, jb, ba, bb, bc, sa, sb, sc):
        base = base_w + c * CHUNK
        pltpu.make_async_copy(a_hbm.at[ib], ba, sa).wait()
        pltpu.make_async_copy(b_hbm.at[jb], bb, sb).wait()
        pltpu.make_async_copy(c_hbm.at[pl.ds(base, CHUNK)], bc, sc).wait()

    def extract_groups(ib, gb):
        def g_body(q, qcarry):
            s = pl.ds(q * 16, 16)
            gb[s] = lax.shift_right_logical(ib[s] * 5243, 19)
            return qcarry
        lax.fori_loop(0, CHUNK // 16, g_body, 0)

    def compute(gb, ba, bb, bc):
        def q_body(q, qcarry):
            gv = gb[pl.ds(q * 16, 16)]
            for l in range(16):
                e = q * 16 + l
                g = gv[l]
                for k in range(8):
                    s = pl.ds(k * 16, 16)
                    v = ba[e, s] + bb[e, s] + bc[e, s]
                    plsc.addupdate(acc_t.at[g, s], jnp.maximum(v, 0.0))
            return qcarry
        lax.fori_loop(0, CHUNK // 16, q_body, 0)

    def step(c, cur, nxt):
        (ci, cj, cg, ca, cb, cc, csa, csb, csc, csi, csj) = cur
        (ni, nj, ng, na, nb, nc, nsa, nsb, nsc, nsi, nsj) = nxt

        @pl.when(c + 1 < CPT)
        def _():
            wait_idx(c + 1, ni, nj, nsi, nsj)
            launch_gather(c + 1, ni, nj, na, nb, nc, nsa, nsb, nsc)

        extract_groups(ci, cg)
        wait_gather(c, ci, cj, ca, cb, cc, csa, csb, csc)

        @pl.when(c + 2 < CPT)
        def _():
            launch_idx(c + 2, ci, cj, csi, csj)

        compute(cg, ca, cb, cc)

    p0 = (i0, j0, g0, a0, b0, c0, sa0, sb0, sc0, si0, sj0)
    p1 = (i1, j1, g1, a1, b1, c1, sa1, sb1, sc1, si1, sj1)

    pltpu.sync_copy(ii_hbm.at[pl.ds(base_w, CHUNK)], i0)
    pltpu.sync_copy(jj_hbm.at[pl.ds(base_w, CHUNK)], j0)
    launch_gather(0, i0, j0, a0, b0, c0, sa0, sb0, sc0)
    launch_idx(1, i1, j1, si1, sj1)

    def pair_body(k, carry):
        step(2 * k, p0, p1)
        step(2 * k + 1, p1, p0)
        return carry
    lax.fori_loop(0, CPT // 2, pair_body, 0)

    pltpu.sync_copy(acc_t, out_hbm.at[cid, tid])


@functools.lru_cache(maxsize=1)
def _sc_edge_kernel():
    return functools.partial(
        pl.kernel,
        out_type=jax.ShapeDtypeStruct((2, 16, ACC_ROWS, F), jnp.float32),
        mesh=plsc.VectorSubcoreMesh(core_axis_name="c", subcore_axis_name="s"),
        scratch_types=[
            pltpu.VMEM((CHUNK,), jnp.int32),
            pltpu.VMEM((CHUNK,), jnp.int32),
            pltpu.VMEM((CHUNK,), jnp.int32),
            pltpu.VMEM((CHUNK,), jnp.int32),
            pltpu.VMEM((CHUNK,), jnp.int32),
            pltpu.VMEM((CHUNK,), jnp.int32),
            pltpu.VMEM((CHUNK, F), jnp.float32),
            pltpu.VMEM((CHUNK, F), jnp.float32),
            pltpu.VMEM((CHUNK, F), jnp.float32),
            pltpu.VMEM((CHUNK, F), jnp.float32),
            pltpu.VMEM((CHUNK, F), jnp.float32),
            pltpu.VMEM((CHUNK, F), jnp.float32),
            pltpu.VMEM((ACC_ROWS, F), jnp.float32),
            pltpu.SemaphoreType.DMA,
            pltpu.SemaphoreType.DMA,
            pltpu.SemaphoreType.DMA,
            pltpu.SemaphoreType.DMA,
            pltpu.SemaphoreType.DMA,
            pltpu.SemaphoreType.DMA,
            pltpu.SemaphoreType.DMA,
            pltpu.SemaphoreType.DMA,
            pltpu.SemaphoreType.DMA,
            pltpu.SemaphoreType.DMA,
        ],
    )(_sc_edge_body)


@jax.jit
def kernel(x, edge_index, edge_attr, W1, b1, W2, b2, lin1_W, lin1_b, gv_W, gv_b):
    f32 = jnp.float32
    idx_i = edge_index[0].astype(jnp.int32)
    idx_j = edge_index[1].astype(jnp.int32)
    npad = EDGES_PAD - N_EDGES
    ii = jnp.concatenate([idx_i, jnp.full((npad,), N_NODES, jnp.int32)])
    jj = jnp.concatenate([idx_j, jnp.zeros((npad,), jnp.int32)])
    gi2d = ii.reshape(EDGES_PAD // 128, 128)
    bf16 = jnp.bfloat16
    ea_pad = jnp.concatenate([edge_attr, jnp.zeros((npad, EA), f32)]).astype(bf16)
    x_pad = jnp.concatenate([x, jnp.zeros((NODES_PAD - N_NODES, F), f32)]).astype(bf16)

    W1a = W1[:, :F].astype(bf16)
    W1b = W1[:, F:2 * F].astype(bf16)
    W1c = W1[:, 2 * F:].astype(bf16)
    b1r = b1.reshape(1, F)

    a_tab, b_tab = pl.pallas_call(
        _t1_body,
        grid=(10,),
        in_specs=[
            pl.BlockSpec((1024, F), lambda i: (i, 0)),
            pl.BlockSpec((F, F), lambda i: (0, 0)),
            pl.BlockSpec((F, F), lambda i: (0, 0)),
        ],
        out_specs=[
            pl.BlockSpec((1024, F), lambda i: (i, 0)),
            pl.BlockSpec((1024, F), lambda i: (i, 0)),
        ],
        out_shape=[
            jax.ShapeDtypeStruct((NODES_PAD, F), f32),
            jax.ShapeDtypeStruct((NODES_PAD, F), f32),
        ],
    )(x_pad, W1a, W1b)

    c_tab, deg = pl.pallas_call(
        _t2_body,
        grid=(EDGES_PAD // 1024,),
        in_specs=[
            pl.BlockSpec((1024, EA), lambda i: (i, 0)),
            pl.BlockSpec((F, EA), lambda i: (0, 0)),
            pl.BlockSpec((1, F), lambda i: (0, 0)),
            pl.BlockSpec((8, 128), lambda i: (i, 0)),
        ],
        out_specs=[
            pl.BlockSpec((1024, F), lambda i: (i, 0)),
            pl.BlockSpec((128, 1), lambda i: (0, 0)),
        ],
        out_shape=[
            jax.ShapeDtypeStruct((EDGES_PAD, F), f32),
            jax.ShapeDtypeStruct((128, 1), f32),
        ],
    )(ea_pad, W1c, b1r, gi2d)

    acc = _sc_edge_kernel()(a_tab, b_tab, c_tab, ii, jj)
    acc32 = acc.reshape(NW, ACC_ROWS, F)

    x3 = x.reshape(100, 100, F)
    out = pl.pallas_call(
        _t4_body,
        in_specs=[pl.BlockSpec(memory_space=pltpu.MemorySpace.VMEM)] * 9
        + [pl.BlockSpec(memory_space=pltpu.MemorySpace.SMEM)],
        out_specs=pl.BlockSpec(memory_space=pltpu.MemorySpace.VMEM),
        out_shape=jax.ShapeDtypeStruct((100, 8), f32),
    )(acc32, deg, x3, W2.astype(bf16).astype(f32), b2.reshape(1, F),
      lin1_W[:, :F], lin1_W[:, F:],
      lin1_b.reshape(1, F), jnp.concatenate([gv_W, jnp.zeros((7, F), f32)]),
      gv_b.reshape(1, 1))

    return out[:, 0]


# speedup vs baseline: 3.3554x; 1.0816x over previous
import functools

import jax
import jax.numpy as jnp
from jax import lax
from jax.experimental import pallas as pl
from jax.experimental.pallas import tpu as pltpu
from jax.experimental.pallas import tpu_sc as plsc

N_NODES = 10000
N_EDGES = 320000
F = 128
EA = 16

NW = 32
CHUNK = 128
CPT = 80
EDGES_PAD = NW * CPT * CHUNK
TILE_EDGES = CPT * CHUNK
NODES_PAD = 10240
ACC_ROWS = 104


def _t1_body(x_ref, wa_ref, wb_ref, a_ref, b_ref):
    xv = x_ref[...]
    dn = (((1,), (1,)), ((), ()))
    a_ref[...] = lax.dot_general(xv, wa_ref[...], dn,
                                 preferred_element_type=jnp.float32)
    b_ref[...] = lax.dot_general(xv, wb_ref[...], dn,
                                 preferred_element_type=jnp.float32)


def _t2_body(ea_ref, wc_ref, b1_ref, gi_ref, c_ref, deg_ref):
    dn = (((1,), (1,)), ((), ()))
    c_ref[...] = lax.dot_general(ea_ref[...], wc_ref[...], dn,
                                 preferred_element_type=jnp.float32) + b1_ref[...]
    gi = gi_ref[...].astype(jnp.float32)
    g = jnp.floor((gi + 0.5) * 0.01).astype(jnp.int32)
    oh = (lax.broadcasted_iota(jnp.int32, (128, 8, 128), 0)
          == g[None, :, :]).astype(jnp.float32)
    part = jnp.sum(oh, axis=(1, 2)).reshape(128, 1)

    @pl.when(pl.program_id(0) == 0)
    def _():
        deg_ref[...] = part

    @pl.when(pl.program_id(0) != 0)
    def _():
        deg_ref[...] = deg_ref[...] + part


def _t4_body(acc_ref, deg_ref, x3_ref, w2_ref, b2_ref, l1a_ref, l1b_ref,
             l1bias_ref, gvw_ref, gvb_ref, out_ref):
    acc = jnp.sum(acc_ref[...], axis=0)
    rsum = acc[:100, :]
    deg = deg_ref[...][:100, :]
    dn = (((1,), (1,)), ((), ()))
    g = lax.dot_general(rsum, w2_ref[...], dn,
                        preferred_element_type=jnp.float32) + deg * b2_ref[...]
    xs = jnp.sum(x3_ref[...], axis=1)
    v = lax.dot_general(xs, l1a_ref[...], dn, preferred_element_type=jnp.float32)
    v = v + lax.dot_general(g, l1b_ref[...], dn, preferred_element_type=jnp.float32)
    v = jnp.maximum(v + l1bias_ref[...], 0.0)
    out_ref[...] = lax.dot_general(v, gvw_ref[...], dn,
                                   preferred_element_type=jnp.float32) + gvb_ref[0, 0]


def _sc_edge_body(a_hbm, b_hbm, c_hbm, ii_hbm, jj_hbm, out_hbm,
                  i0, j0, i1, j1, g0, g1, a0, b0, c0, a1, b1, c1, zbuf,
                  acc_sh, sa0, sb0, sc0, sa1, sb1, sc1, si0, sj0, si1, sj1):
    cid = lax.axis_index("c")
    tid = lax.axis_index("s")
    wid = cid * 16 + tid
    base_w = wid * TILE_EDGES

    zero16 = jnp.zeros((16,), jnp.float32)

    @pl.when(tid == 0)
    def _():
        def zrow(r, carry):
            for k in range(8):
                zbuf[r, pl.ds(k * 16, 16)] = zero16
            return carry
        lax.fori_loop(0, ACC_ROWS, zrow, 0)
        pltpu.sync_copy(zbuf, acc_sh)
    plsc.subcore_barrier()

    def launch_idx(c, ib, jb, si, sj):
        base = base_w + c * CHUNK
        pltpu.async_copy(ii_hbm.at[pl.ds(base, CHUNK)], ib, si)
        pltpu.async_copy(jj_hbm.at[pl.ds(base, CHUNK)], jb, sj)

    def wait_idx(c, ib, jb, si, sj):
        base = base_w + c * CHUNK
        pltpu.make_async_copy(ii_hbm.at[pl.ds(base, CHUNK)], ib, si).wait()
        pltpu.make_async_copy(jj_hbm.at[pl.ds(base, CHUNK)], jb, sj).wait()

    def launch_gather(c, ib, jb, ba, bb, bc, sa, sb, sc):
        base = base_w + c * CHUNK
        pltpu.async_copy(a_hbm.at[ib], ba, sa)
        pltpu.async_copy(b_hbm.at[jb], bb, sb)
        pltpu.async_copy(c_hbm.at[pl.ds(base, CHUNK)], bc, sc)

    def wait_gather(c, ib, jb, ba, bb, bc, sa, sb, sc):
        base = base_w + c * CHUNK
        pltpu.make_async_copy(a_hbm.at[ib], ba, sa).wait()
        pltpu.make_async_copy(b_hbm.at[jb], bb, sb).wait()
        pltpu.make_async_copy(c_hbm.at[pl.ds(base, CHUNK)], bc, sc).wait()

    def extract_groups(ib, gb):
        def g_body(q, qcarry):
            s = pl.ds(q * 16, 16)
            gb[s] = lax.shift_right_logical(ib[s] * 5243, 19)
            return qcarry
        lax.fori_loop(0, CHUNK // 16, g_body, 0)

    def compute_m(ba, bb, bc):
        def q_body(e, qcarry):
            for k in range(8):
                s = pl.ds(k * 16, 16)
                v = ba[e, s] + bb[e, s] + bc[e, s]
                ba[e, s] = jnp.maximum(v, 0.0)
            return qcarry
        lax.fori_loop(0, CHUNK, q_body, 0)

    def step(c, cur, nxt):
        (ci, cj, cg, ca, cb, cc, csa, csb, csc, csi, csj) = cur
        (ni, nj, ng, na, nb, nc, nsa, nsb, nsc, nsi, nsj) = nxt

        @pl.when(c + 1 < CPT)
        def _():
            wait_idx(c + 1, ni, nj, nsi, nsj)
            launch_gather(c + 1, ni, nj, na, nb, nc, nsa, nsb, nsc)

        extract_groups(ci, cg)
        wait_gather(c, ci, cj, ca, cb, cc, csa, csb, csc)

        @pl.when(c + 2 < CPT)
        def _():
            launch_idx(c + 2, ci, cj, csi, csj)

        compute_m(ca, cb, cc)
        pltpu.sync_copy(ca, acc_sh.at[cg], add=True)

    p0 = (i0, j0, g0, a0, b0, c0, sa0, sb0, sc0, si0, sj0)
    p1 = (i1, j1, g1, a1, b1, c1, sa1, sb1, sc1, si1, sj1)

    pltpu.sync_copy(ii_hbm.at[pl.ds(base_w, CHUNK)], i0)
    pltpu.sync_copy(jj_hbm.at[pl.ds(base_w, CHUNK)], j0)
    launch_gather(0, i0, j0, a0, b0, c0, sa0, sb0, sc0)
    launch_idx(1, i1, j1, si1, sj1)

    def pair_body(k, carry):
        step(2 * k, p0, p1)
        step(2 * k + 1, p1, p0)
        return carry
    lax.fori_loop(0, CPT // 2, pair_body, 0)

    plsc.subcore_barrier()

    @pl.when(tid == 0)
    def _():
        pltpu.sync_copy(acc_sh, out_hbm.at[cid])


@functools.lru_cache(maxsize=1)
def _sc_edge_kernel():
    return functools.partial(
        pl.kernel,
        out_type=jax.ShapeDtypeStruct((2, ACC_ROWS, F), jnp.float32),
        mesh=plsc.VectorSubcoreMesh(core_axis_name="c", subcore_axis_name="s"),
        scratch_types=[
            pltpu.VMEM((CHUNK,), jnp.int32),
            pltpu.VMEM((CHUNK,), jnp.int32),
            pltpu.VMEM((CHUNK,), jnp.int32),
            pltpu.VMEM((CHUNK,), jnp.int32),
            pltpu.VMEM((CHUNK,), jnp.int32),
            pltpu.VMEM((CHUNK,), jnp.int32),
            pltpu.VMEM((CHUNK, F), jnp.float32),
            pltpu.VMEM((CHUNK, F), jnp.float32),
            pltpu.VMEM((CHUNK, F), jnp.float32),
            pltpu.VMEM((CHUNK, F), jnp.float32),
            pltpu.VMEM((CHUNK, F), jnp.float32),
            pltpu.VMEM((CHUNK, F), jnp.float32),
            pltpu.VMEM((ACC_ROWS, F), jnp.float32),
            pltpu.VMEM_SHARED((ACC_ROWS, F), jnp.float32),
            pltpu.SemaphoreType.DMA,
            pltpu.SemaphoreType.DMA,
            pltpu.SemaphoreType.DMA,
            pltpu.SemaphoreType.DMA,
            pltpu.SemaphoreType.DMA,
            pltpu.SemaphoreType.DMA,
            pltpu.SemaphoreType.DMA,
            pltpu.SemaphoreType.DMA,
            pltpu.SemaphoreType.DMA,
            pltpu.SemaphoreType.DMA,
        ],
    )(_sc_edge_body)


@jax.jit
def kernel(x, edge_index, edge_attr, W1, b1, W2, b2, lin1_W, lin1_b, gv_W, gv_b):
    f32 = jnp.float32
    idx_i = edge_index[0].astype(jnp.int32)
    idx_j = edge_index[1].astype(jnp.int32)
    npad = EDGES_PAD - N_EDGES
    ii = jnp.concatenate([idx_i, jnp.full((npad,), N_NODES, jnp.int32)])
    jj = jnp.concatenate([idx_j, jnp.zeros((npad,), jnp.int32)])
    gi2d = ii.reshape(EDGES_PAD // 128, 128)
    bf16 = jnp.bfloat16
    ea_pad = jnp.concatenate([edge_attr, jnp.zeros((npad, EA), f32)]).astype(bf16)
    x_pad = jnp.concatenate([x, jnp.zeros((NODES_PAD - N_NODES, F), f32)]).astype(bf16)

    W1a = W1[:, :F].astype(bf16)
    W1b = W1[:, F:2 * F].astype(bf16)
    W1c = W1[:, 2 * F:].astype(bf16)
    b1r = b1.reshape(1, F)

    a_tab, b_tab = pl.pallas_call(
        _t1_body,
        grid=(10,),
        in_specs=[
            pl.BlockSpec((1024, F), lambda i: (i, 0)),
            pl.BlockSpec((F, F), lambda i: (0, 0)),
            pl.BlockSpec((F, F), lambda i: (0, 0)),
        ],
        out_specs=[
            pl.BlockSpec((1024, F), lambda i: (i, 0)),
            pl.BlockSpec((1024, F), lambda i: (i, 0)),
        ],
        out_shape=[
            jax.ShapeDtypeStruct((NODES_PAD, F), f32),
            jax.ShapeDtypeStruct((NODES_PAD, F), f32),
        ],
    )(x_pad, W1a, W1b)

    c_tab, deg = pl.pallas_call(
        _t2_body,
        grid=(EDGES_PAD // 1024,),
        in_specs=[
            pl.BlockSpec((1024, EA), lambda i: (i, 0)),
            pl.BlockSpec((F, EA), lambda i: (0, 0)),
            pl.BlockSpec((1, F), lambda i: (0, 0)),
            pl.BlockSpec((8, 128), lambda i: (i, 0)),
        ],
        out_specs=[
            pl.BlockSpec((1024, F), lambda i: (i, 0)),
            pl.BlockSpec((128, 1), lambda i: (0, 0)),
        ],
        out_shape=[
            jax.ShapeDtypeStruct((EDGES_PAD, F), f32),
            jax.ShapeDtypeStruct((128, 1), f32),
        ],
    )(ea_pad, W1c, b1r, gi2d)

    acc32 = _sc_edge_kernel()(a_tab, b_tab, c_tab, ii, jj)

    x3 = x.reshape(100, 100, F)
    out = pl.pallas_call(
        _t4_body,
        in_specs=[pl.BlockSpec(memory_space=pltpu.MemorySpace.VMEM)] * 9
        + [pl.BlockSpec(memory_space=pltpu.MemorySpace.SMEM)],
        out_specs=pl.BlockSpec(memory_space=pltpu.MemorySpace.VMEM),
        out_shape=jax.ShapeDtypeStruct((100, 8), f32),
    )(acc32, deg, x3, W2.astype(bf16).astype(f32), b2.reshape(1, F),
      lin1_W[:, :F], lin1_W[:, F:],
      lin1_b.reshape(1, F), jnp.concatenate([gv_W, jnp.zeros((7, F), f32)]),
      gv_b.reshape(1, 1))

    return out[:, 0]


# speedup vs baseline: 3.3591x; 1.0011x over previous
import functools

import jax
import jax.numpy as jnp
from jax import lax
from jax.experimental import pallas as pl
from jax.experimental.pallas import tpu as pltpu
from jax.experimental.pallas import tpu_sc as plsc

N_NODES = 10000
N_EDGES = 320000
F = 128
EA = 16

NW = 32
CHUNK = 128
CPT = 80
EDGES_PAD = NW * CPT * CHUNK
TILE_EDGES = CPT * CHUNK
NODES_PAD = 10240
ACC_ROWS = 104


def _t1_body(x_ref, wa_ref, wb_ref, a_ref, b_ref):
    xv = x_ref[...]
    dn = (((1,), (1,)), ((), ()))
    a_ref[...] = lax.dot_general(xv, wa_ref[...], dn,
                                 preferred_element_type=jnp.float32)
    b_ref[...] = lax.dot_general(xv, wb_ref[...], dn,
                                 preferred_element_type=jnp.float32)


def _t2_body(ea_ref, wc_ref, b1_ref, gi_ref, c_ref, deg_ref):
    dn = (((1,), (1,)), ((), ()))
    c_ref[...] = lax.dot_general(ea_ref[...], wc_ref[...], dn,
                                 preferred_element_type=jnp.float32) + b1_ref[...]
    gi = gi_ref[...].astype(jnp.float32)
    g = jnp.floor((gi + 0.5) * 0.01).astype(jnp.int32)
    oh = (lax.broadcasted_iota(jnp.int32, (128, 8, 128), 0)
          == g[None, :, :]).astype(jnp.float32)
    part = jnp.sum(oh, axis=(1, 2)).reshape(128, 1)

    @pl.when(pl.program_id(0) == 0)
    def _():
        deg_ref[...] = part

    @pl.when(pl.program_id(0) != 0)
    def _():
        deg_ref[...] = deg_ref[...] + part


def _t4_body(acc_ref, deg_ref, x3_ref, w2_ref, b2_ref, l1a_ref, l1b_ref,
             l1bias_ref, gvw_ref, gvb_ref, out_ref):
    acc = jnp.sum(acc_ref[...], axis=0)
    rsum = acc[:100, :]
    deg = deg_ref[...][:100, :]
    dn = (((1,), (1,)), ((), ()))
    g = lax.dot_general(rsum, w2_ref[...], dn,
                        preferred_element_type=jnp.float32) + deg * b2_ref[...]
    xs = jnp.sum(x3_ref[...], axis=1)
    v = lax.dot_general(xs, l1a_ref[...], dn, preferred_element_type=jnp.float32)
    v = v + lax.dot_general(g, l1b_ref[...], dn, preferred_element_type=jnp.float32)
    v = jnp.maximum(v + l1bias_ref[...], 0.0)
    out_ref[...] = lax.dot_general(v, gvw_ref[...], dn,
                                   preferred_element_type=jnp.float32) + gvb_ref[0, 0]


def _sc_edge_body(a_hbm, b_hbm, c_hbm, ii_hbm, jj_hbm, out_hbm,
                  i0, j0, i1, j1, g0, g1, a0, b0, c0, a1, b1, c1, zbuf,
                  acc_sh, sa0, sb0, sc0, sa1, sb1, sc1, si0, sj0, si1, sj1,
                  ss0, ss1):
    cid = lax.axis_index("c")
    tid = lax.axis_index("s")
    wid = cid * 16 + tid
    base_w = wid * TILE_EDGES

    zero16 = jnp.zeros((16,), jnp.float32)

    @pl.when(tid == 0)
    def _():
        def zrow(r, carry):
            for k in range(8):
                zbuf[r, pl.ds(k * 16, 16)] = zero16
            return carry
        lax.fori_loop(0, ACC_ROWS, zrow, 0)
        pltpu.sync_copy(zbuf, acc_sh)
    plsc.subcore_barrier()

    def launch_idx(c, ib, jb, si, sj):
        base = base_w + c * CHUNK
        pltpu.async_copy(ii_hbm.at[pl.ds(base, CHUNK)], ib, si)
        pltpu.async_copy(jj_hbm.at[pl.ds(base, CHUNK)], jb, sj)

    def wait_idx(c, ib, jb, si, sj):
        base = base_w + c * CHUNK
        pltpu.make_async_copy(ii_hbm.at[pl.ds(base, CHUNK)], ib, si).wait()
        pltpu.make_async_copy(jj_hbm.at[pl.ds(base, CHUNK)], jb, sj).wait()

    def launch_gather(c, ib, jb, ba, bb, bc, sa, sb, sc):
        base = base_w + c * CHUNK
        pltpu.async_copy(a_hbm.at[ib], ba, sa)
        pltpu.async_copy(b_hbm.at[jb], bb, sb)
        pltpu.async_copy(c_hbm.at[pl.ds(base, CHUNK)], bc, sc)

    def wait_gather(c, ib, jb, ba, bb, bc, sa, sb, sc):
        base = base_w + c * CHUNK
        pltpu.make_async_copy(a_hbm.at[ib], ba, sa).wait()
        pltpu.make_async_copy(b_hbm.at[jb], bb, sb).wait()
        pltpu.make_async_copy(c_hbm.at[pl.ds(base, CHUNK)], bc, sc).wait()

    def extract_groups(ib, gb):
        def g_body(q, qcarry):
            s = pl.ds(q * 16, 16)
            gb[s] = lax.shift_right_logical(ib[s] * 5243, 19)
            return qcarry
        lax.fori_loop(0, CHUNK // 16, g_body, 0)

    def compute_m(ba, bb, bc):
        def q_body(e, qcarry):
            for k in range(8):
                s = pl.ds(k * 16, 16)
                v = ba[e, s] + bb[e, s] + bc[e, s]
                ba[e, s] = jnp.maximum(v, 0.0)
            return qcarry
        lax.fori_loop(0, CHUNK, q_body, 0)

    def step(c, cur, nxt):
        (ci, cj, cg, ca, cb, cc, csa, csb, csc, csi, csj, css) = cur
        (ni, nj, ng, na, nb, nc, nsa, nsb, nsc, nsi, nsj, nss) = nxt

        @pl.when(c + 1 < CPT)
        def _():
            wait_idx(c + 1, ni, nj, nsi, nsj)

            @pl.when(c >= 1)
            def _():
                pltpu.make_async_copy(na, acc_sh.at[ng], nss).wait()
            launch_gather(c + 1, ni, nj, na, nb, nc, nsa, nsb, nsc)

        extract_groups(ci, cg)
        wait_gather(c, ci, cj, ca, cb, cc, csa, csb, csc)

        @pl.when(c + 2 < CPT)
        def _():
            launch_idx(c + 2, ci, cj, csi, csj)

        compute_m(ca, cb, cc)
        pltpu.async_copy(ca, acc_sh.at[cg], css, add=True)

    p0 = (i0, j0, g0, a0, b0, c0, sa0, sb0, sc0, si0, sj0, ss0)
    p1 = (i1, j1, g1, a1, b1, c1, sa1, sb1, sc1, si1, sj1, ss1)

    pltpu.sync_copy(ii_hbm.at[pl.ds(base_w, CHUNK)], i0)
    pltpu.sync_copy(jj_hbm.at[pl.ds(base_w, CHUNK)], j0)
    launch_gather(0, i0, j0, a0, b0, c0, sa0, sb0, sc0)
    launch_idx(1, i1, j1, si1, sj1)

    def pair_body(k, carry):
        step(2 * k, p0, p1)
        step(2 * k + 1, p1, p0)
        return carry
    lax.fori_loop(0, CPT // 2, pair_body, 0)

    pltpu.make_async_copy(a0, acc_sh.at[g0], ss0).wait()
    pltpu.make_async_copy(a1, acc_sh.at[g1], ss1).wait()

    plsc.subcore_barrier()

    @pl.when(tid == 0)
    def _():
        pltpu.sync_copy(acc_sh, out_hbm.at[cid])


@functools.lru_cache(maxsize=1)
def _sc_edge_kernel():
    return functools.partial(
        pl.kernel,
        out_type=jax.ShapeDtypeStruct((2, ACC_ROWS, F), jnp.float32),
        mesh=plsc.VectorSubcoreMesh(core_axis_name="c", subcore_axis_name="s"),
        scratch_types=[
            pltpu.VMEM((CHUNK,), jnp.int32),
            pltpu.VMEM((CHUNK,), jnp.int32),
            pltpu.VMEM((CHUNK,), jnp.int32),
            pltpu.VMEM((CHUNK,), jnp.int32),
            pltpu.VMEM((CHUNK,), jnp.int32),
            pltpu.VMEM((CHUNK,), jnp.int32),
            pltpu.VMEM((CHUNK, F), jnp.float32),
            pltpu.VMEM((CHUNK, F), jnp.float32),
            pltpu.VMEM((CHUNK, F), jnp.float32),
            pltpu.VMEM((CHUNK, F), jnp.float32),
            pltpu.VMEM((CHUNK, F), jnp.float32),
            pltpu.VMEM((CHUNK, F), jnp.float32),
            pltpu.VMEM((ACC_ROWS, F), jnp.float32),
            pltpu.VMEM_SHARED((ACC_ROWS, F), jnp.float32),
            pltpu.SemaphoreType.DMA,
            pltpu.SemaphoreType.DMA,
            pltpu.SemaphoreType.DMA,
            pltpu.SemaphoreType.DMA,
            pltpu.SemaphoreType.DMA,
            pltpu.SemaphoreType.DMA,
            pltpu.SemaphoreType.DMA,
            pltpu.SemaphoreType.DMA,
            pltpu.SemaphoreType.DMA,
            pltpu.SemaphoreType.DMA,
            pltpu.SemaphoreType.DMA,
            pltpu.SemaphoreType.DMA,
        ],
    )(_sc_edge_body)


@jax.jit
def kernel(x, edge_index, edge_attr, W1, b1, W2, b2, lin1_W, lin1_b, gv_W, gv_b):
    f32 = jnp.float32
    idx_i = edge_index[0].astype(jnp.int32)
    idx_j = edge_index[1].astype(jnp.int32)
    npad = EDGES_PAD - N_EDGES
    ii = jnp.concatenate([idx_i, jnp.full((npad,), N_NODES, jnp.int32)])
    jj = jnp.concatenate([idx_j, jnp.zeros((npad,), jnp.int32)])
    gi2d = ii.reshape(EDGES_PAD // 128, 128)
    bf16 = jnp.bfloat16
    ea_pad = jnp.concatenate([edge_attr, jnp.zeros((npad, EA), f32)]).astype(bf16)
    x_pad = jnp.concatenate([x, jnp.zeros((NODES_PAD - N_NODES, F), f32)]).astype(bf16)

    W1a = W1[:, :F].astype(bf16)
    W1b = W1[:, F:2 * F].astype(bf16)
    W1c = W1[:, 2 * F:].astype(bf16)
    b1r = b1.reshape(1, F)

    a_tab, b_tab = pl.pallas_call(
        _t1_body,
        grid=(10,),
        in_specs=[
            pl.BlockSpec((1024, F), lambda i: (i, 0)),
            pl.BlockSpec((F, F), lambda i: (0, 0)),
            pl.BlockSpec((F, F), lambda i: (0, 0)),
        ],
        out_specs=[
            pl.BlockSpec((1024, F), lambda i: (i, 0)),
            pl.BlockSpec((1024, F), lambda i: (i, 0)),
        ],
        out_shape=[
            jax.ShapeDtypeStruct((NODES_PAD, F), f32),
            jax.ShapeDtypeStruct((NODES_PAD, F), f32),
        ],
    )(x_pad, W1a, W1b)

    c_tab, deg = pl.pallas_call(
        _t2_body,
        grid=(EDGES_PAD // 1024,),
        in_specs=[
            pl.BlockSpec((1024, EA), lambda i: (i, 0)),
            pl.BlockSpec((F, EA), lambda i: (0, 0)),
            pl.BlockSpec((1, F), lambda i: (0, 0)),
            pl.BlockSpec((8, 128), lambda i: (i, 0)),
        ],
        out_specs=[
            pl.BlockSpec((1024, F), lambda i: (i, 0)),
            pl.BlockSpec((128, 1), lambda i: (0, 0)),
        ],
        out_shape=[
            jax.ShapeDtypeStruct((EDGES_PAD, F), f32),
            jax.ShapeDtypeStruct((128, 1), f32),
        ],
    )(ea_pad, W1c, b1r, gi2d)

    acc32 = _sc_edge_kernel()(a_tab, b_tab, c_tab, ii, jj)

    x3 = x.reshape(100, 100, F)
    out = pl.pallas_call(
        _t4_body,
        in_specs=[pl.BlockSpec(memory_space=pltpu.MemorySpace.VMEM)] * 9
        + [pl.BlockSpec(memory_space=pltpu.MemorySpace.SMEM)],
        out_specs=pl.BlockSpec(memory_space=pltpu.MemorySpace.VMEM),
        out_shape=jax.ShapeDtypeStruct((100, 8), f32),
    )(acc32, deg, x3, W2.astype(bf16).astype(f32), b2.reshape(1, F),
      lin1_W[:, :F], lin1_W[:, F:],
      lin1_b.reshape(1, F), jnp.concatenate([gv_W, jnp.zeros((7, F), f32)]),
      gv_b.reshape(1, 1))

    return out[:, 0]
